# Initial kernel scaffold; baseline (speedup 1.0000x reference)
import jax
import jax.numpy as jnp
from jax.experimental import pallas as pl


def kernel(positions, weights, log_scales):
    raise NotImplementedError("write your pallas kernel here")


# speedup vs baseline: 49.0141x; 49.0141x over previous
import functools

import jax
import jax.numpy as jnp
from jax import lax
from jax.experimental import pallas as pl
from jax.experimental.pallas import tpu as pltpu
from jax.experimental.pallas import tpu_sc as plsc

N = 8192
K = 5
LAMBDA_S = 0.01
ROWS = 256
BIGF = 1e9


_IDX_MASK = 0x1FFF
_BIG_KEY = 0x7F000000


def _topk_body(pos_blk, pos_t, out_idx, key_ref):
    xs = pos_blk[:, 0:1]
    ys = pos_blk[:, 1:2]
    zs = pos_blk[:, 2:3]
    xc = pos_t[0:1, :]
    yc = pos_t[1:2, :]
    zc = pos_t[2:3, :]
    bf = jnp.bfloat16
    f32 = jnp.float32
    xsb = xs.astype(bf).astype(f32)
    ysb = ys.astype(bf).astype(f32)
    zsb = zs.astype(bf).astype(f32)
    xcb = xc.astype(bf).astype(f32)
    ycb = yc.astype(bf).astype(f32)
    zcb = zc.astype(bf).astype(f32)
    dot = xsb * xcb + ysb * ycb + zsb * zcb
    sq_r = xs * xs + ys * ys + zs * zs
    sq_c = xc * xc + yc * yc + zc * zc
    d2 = sq_r + sq_c - 2.0 * dot
    d2 = jnp.maximum(d2, 0.0)

    col = lax.broadcasted_iota(jnp.int32, (ROWS, N), 1)
    bits = lax.bitcast_convert_type(d2, jnp.int32)
    key_ref[...] = (bits & jnp.int32(~_IDX_MASK)) | col
    for j in range(K + 1):
        key = key_ref[...]
        m = jnp.min(key, axis=1, keepdims=True)
        if j > 0:
            out_idx[:, j - 1 : j] = m & _IDX_MASK
        if j < K:
            key_ref[...] = jnp.where(key == m, jnp.int32(_BIG_KEY), key)


_NC = 2
_NS = 16
_NW = _NC * _NS
_PER = (N * K) // _NW
_CH = _PER // 16

@functools.cache
def _make_sc_loss():
    mesh = plsc.VectorSubcoreMesh(core_axis_name="c", subcore_axis_name="s")

    @functools.partial(
        pl.kernel,
        mesh=mesh,
        compiler_params=pltpu.CompilerParams(needs_layout_passes=False),
        out_type=jax.ShapeDtypeStruct((_NW, 2, 16), jnp.float32),
        scratch_types=[
            pltpu.VMEM((N,), jnp.float32),
            pltpu.VMEM((N,), jnp.float32),
            pltpu.VMEM((N,), jnp.float32),
            pltpu.VMEM((N,), jnp.float32),
            pltpu.VMEM((_PER,), jnp.int32),
            pltpu.VMEM((_PER,), jnp.int32),
            pltpu.VMEM((16,), jnp.float32),
            pltpu.VMEM((16,), jnp.float32),
        ],
    )
    def _sc_loss(w_hbm, lx_hbm, ly_hbm, lz_hbm, idxn_hbm, idxs_hbm, out_hbm,
                 w_v, x_v, y_v, z_v, in_v, is_v, ow_v, ol_v):
        wid = lax.axis_index("s") * _NC + lax.axis_index("c")
        base = wid * _PER
        pltpu.sync_copy(w_hbm, w_v)
        pltpu.sync_copy(lx_hbm, x_v)
        pltpu.sync_copy(ly_hbm, y_v)
        pltpu.sync_copy(lz_hbm, z_v)
        pltpu.sync_copy(idxn_hbm.at[pl.ds(base, _PER)], in_v)
        pltpu.sync_copy(idxs_hbm.at[pl.ds(base, _PER)], is_v)

        def body(t, carry):
            accw, accl = carry
            off = t * 16
            jn = in_v[pl.ds(off, 16)]
            js = is_v[pl.ds(off, 16)]
            wj = plsc.load_gather(w_v, [jn])
            wi = plsc.load_gather(w_v, [js])
            dw = wi - wj
            accw = accw + dw * dw
            xj = plsc.load_gather(x_v, [jn])
            xi = plsc.load_gather(x_v, [js])
            dx = xi - xj
            accl = accl + dx * dx
            yj = plsc.load_gather(y_v, [jn])
            yi = plsc.load_gather(y_v, [js])
            dy = yi - yj
            accl = accl + dy * dy
            zj = plsc.load_gather(z_v, [jn])
            zi = plsc.load_gather(z_v, [js])
            dz = zi - zj
            accl = accl + dz * dz
            return accw, accl

        zero = jnp.zeros((16,), jnp.float32)
        accw, accl = lax.fori_loop(0, _CH, body, (zero, zero))
        ow_v[...] = accw
        ol_v[...] = accl
        pltpu.sync_copy(ow_v, out_hbm.at[wid, 0])
        pltpu.sync_copy(ol_v, out_hbm.at[wid, 1])

    return _sc_loss


def kernel(positions, weights, log_scales):
    idx = pl.pallas_call(
        _topk_body,
        grid=(N // ROWS,),
        in_specs=[
            pl.BlockSpec((ROWS, 3), lambda i: (i, 0)),
            pl.BlockSpec((3, N), lambda i: (0, 0)),
        ],
        out_specs=pl.BlockSpec((ROWS, K), lambda i: (i, 0)),
        out_shape=jax.ShapeDtypeStruct((N, K), jnp.int32),
        scratch_shapes=[pltpu.VMEM((ROWS, N), jnp.int32)],
    )(positions, positions.T)

    idxn = idx.reshape(-1)
    idxs = jnp.repeat(jnp.arange(N, dtype=jnp.int32), K)
    parts = _make_sc_loss()(
        weights,
        log_scales[:, 0],
        log_scales[:, 1],
        log_scales[:, 2],
        idxn,
        idxs,
    )
    sw = jnp.sum(parts[:, 0, :])
    sl = jnp.sum(parts[:, 1, :])
    loss = LAMBDA_S * (sw / (N * K) + sl / (N * K * 3))
    return loss.astype(jnp.float32)


# speedup vs baseline: 69.1034x; 1.4099x over previous
import functools

import jax
import jax.numpy as jnp
from jax import lax
from jax.experimental import pallas as pl
from jax.experimental.pallas import tpu as pltpu
from jax.experimental.pallas import tpu_sc as plsc

N = 8192
K = 5
LAMBDA_S = 0.01
ROWS = 256

_IDX_MASK = 0x1FFF


def _topk_body(pos_blk, pos_t, out_idx, key_ref):
    xs = pos_blk[:, 0:1]
    ys = pos_blk[:, 1:2]
    zs = pos_blk[:, 2:3]
    xc = pos_t[0:1, :]
    yc = pos_t[1:2, :]
    zc = pos_t[2:3, :]
    bf = jnp.bfloat16
    f32 = jnp.float32
    dot = lax.dot_general(
        pos_blk[...].astype(bf),
        pos_t[...].astype(bf),
        (((1,), (0,)), ((), ())),
        preferred_element_type=f32,
    )
    sq_r = xs * xs + ys * ys + zs * zs
    sq_c = xc * xc + yc * yc + zc * zc
    d2 = sq_r + sq_c - 2.0 * dot
    d2 = jnp.maximum(d2, 0.0)

    col = lax.broadcasted_iota(jnp.int32, (ROWS, N), 1)
    bits = lax.bitcast_convert_type(d2, jnp.int32)
    key_ref[...] = lax.bitcast_convert_type(
        ((bits & jnp.int32(~_IDX_MASK)) | col) + jnp.int32(0x00800000), f32
    )
    big = jnp.float32(2.0**127)
    for j in range(K + 1):
        key = key_ref[...]
        m = jnp.min(key, axis=1, keepdims=True)
        if j > 0:
            out_idx[:, j - 1 : j] = (
                lax.bitcast_convert_type(m, jnp.int32) & _IDX_MASK
            )
        if j < K:
            key_ref[...] = jnp.where(key == m, big, key)


_NC = 2
_NS = 16
_NW = _NC * _NS
_PER = (N * K) // _NW
_CH = _PER // 16

@functools.cache
def _make_sc_loss():
    mesh = plsc.VectorSubcoreMesh(core_axis_name="c", subcore_axis_name="s")

    @functools.partial(
        pl.kernel,
        mesh=mesh,
        compiler_params=pltpu.CompilerParams(needs_layout_passes=False),
        out_type=jax.ShapeDtypeStruct((_NW, 2, 16), jnp.float32),
        scratch_types=[
            pltpu.VMEM((N,), jnp.float32),
            pltpu.VMEM((N,), jnp.float32),
            pltpu.VMEM((N,), jnp.float32),
            pltpu.VMEM((N,), jnp.float32),
            pltpu.VMEM((_PER,), jnp.int32),
            pltpu.VMEM((_PER,), jnp.int32),
            pltpu.VMEM((16,), jnp.float32),
            pltpu.VMEM((16,), jnp.float32),
        ],
    )
    def _sc_loss(w_hbm, lx_hbm, ly_hbm, lz_hbm, idxn_hbm, idxs_hbm, out_hbm,
                 w_v, x_v, y_v, z_v, in_v, is_v, ow_v, ol_v):
        wid = lax.axis_index("s") * _NC + lax.axis_index("c")
        base = wid * _PER
        pltpu.sync_copy(w_hbm, w_v)
        pltpu.sync_copy(lx_hbm, x_v)
        pltpu.sync_copy(ly_hbm, y_v)
        pltpu.sync_copy(lz_hbm, z_v)
        pltpu.sync_copy(idxn_hbm.at[pl.ds(base, _PER)], in_v)
        pltpu.sync_copy(idxs_hbm.at[pl.ds(base, _PER)], is_v)

        def body(t, carry):
            accw, accl = carry
            off = t * 16
            jn = in_v[pl.ds(off, 16)]
            js = is_v[pl.ds(off, 16)]
            wj = plsc.load_gather(w_v, [jn])
            wi = plsc.load_gather(w_v, [js])
            dw = wi - wj
            accw = accw + dw * dw
            xj = plsc.load_gather(x_v, [jn])
            xi = plsc.load_gather(x_v, [js])
            dx = xi - xj
            accl = accl + dx * dx
            yj = plsc.load_gather(y_v, [jn])
            yi = plsc.load_gather(y_v, [js])
            dy = yi - yj
            accl = accl + dy * dy
            zj = plsc.load_gather(z_v, [jn])
            zi = plsc.load_gather(z_v, [js])
            dz = zi - zj
            accl = accl + dz * dz
            return accw, accl

        zero = jnp.zeros((16,), jnp.float32)
        accw, accl = lax.fori_loop(0, _CH, body, (zero, zero))
        ow_v[...] = accw
        ol_v[...] = accl
        pltpu.sync_copy(ow_v, out_hbm.at[wid, 0])
        pltpu.sync_copy(ol_v, out_hbm.at[wid, 1])

    return _sc_loss


def kernel(positions, weights, log_scales):
    idx = pl.pallas_call(
        _topk_body,
        grid=(N // ROWS,),
        in_specs=[
            pl.BlockSpec((ROWS, 3), lambda i: (i, 0)),
            pl.BlockSpec((3, N), lambda i: (0, 0)),
        ],
        out_specs=pl.BlockSpec((ROWS, K), lambda i: (i, 0)),
        out_shape=jax.ShapeDtypeStruct((N, K), jnp.int32),
        scratch_shapes=[pltpu.VMEM((ROWS, N), jnp.float32)],
    )(positions, positions.T)

    idxn = idx.reshape(-1)
    idxs = jnp.repeat(jnp.arange(N, dtype=jnp.int32), K)
    parts = _make_sc_loss()(
        weights,
        log_scales[:, 0],
        log_scales[:, 1],
        log_scales[:, 2],
        idxn,
        idxs,
    )
    sw = jnp.sum(parts[:, 0, :])
    sl = jnp.sum(parts[:, 1, :])
    loss = LAMBDA_S * (sw / (N * K) + sl / (N * K * 3))
    return loss.astype(jnp.float32)


# speedup vs baseline: 72.0632x; 1.0428x over previous
import functools

import jax
import jax.numpy as jnp
from jax import lax
from jax.experimental import pallas as pl
from jax.experimental.pallas import tpu as pltpu
from jax.experimental.pallas import tpu_sc as plsc

N = 8192
K = 5
LAMBDA_S = 0.01
ROWS = 512

_IDX_MASK = 0x1FFF


def _topk_body(pos_blk, pos_t, out_idx, key_ref):
    xs = pos_blk[:, 0:1]
    ys = pos_blk[:, 1:2]
    zs = pos_blk[:, 2:3]
    xc = pos_t[0:1, :]
    yc = pos_t[1:2, :]
    zc = pos_t[2:3, :]
    bf = jnp.bfloat16
    f32 = jnp.float32
    dot = lax.dot_general(
        pos_blk[...].astype(bf),
        pos_t[...].astype(bf),
        (((1,), (0,)), ((), ())),
        preferred_element_type=f32,
    )
    sq_r = xs * xs + ys * ys + zs * zs
    sq_c = xc * xc + yc * yc + zc * zc
    d2 = sq_r + sq_c - 2.0 * dot
    d2 = jnp.maximum(d2, 0.0)

    col = lax.broadcasted_iota(jnp.int32, (ROWS, N), 1)
    bits = lax.bitcast_convert_type(d2, jnp.int32)
    key_ref[...] = lax.bitcast_convert_type(
        ((bits & jnp.int32(~_IDX_MASK)) | col) + jnp.int32(0x00800000), f32
    )
    big = jnp.float32(2.0**127)
    for j in range(K + 1):
        key = key_ref[...]
        m = jnp.min(key, axis=1, keepdims=True)
        if j > 0:
            out_idx[:, j - 1 : j] = (
                lax.bitcast_convert_type(m, jnp.int32) & _IDX_MASK
            )
        if j < K:
            key_ref[...] = jnp.where(key == m, big, key)


_NC = 2
_NS = 16
_NW = _NC * _NS
_PER = (N * K) // _NW
_CH = _PER // 16

@functools.cache
def _make_sc_loss():
    mesh = plsc.VectorSubcoreMesh(core_axis_name="c", subcore_axis_name="s")

    @functools.partial(
        pl.kernel,
        mesh=mesh,
        compiler_params=pltpu.CompilerParams(needs_layout_passes=False),
        out_type=jax.ShapeDtypeStruct((_NW, 2, 16), jnp.float32),
        scratch_types=[
            pltpu.VMEM((N,), jnp.float32),
            pltpu.VMEM((N,), jnp.float32),
            pltpu.VMEM((N,), jnp.float32),
            pltpu.VMEM((N,), jnp.float32),
            pltpu.VMEM((_PER,), jnp.int32),
            pltpu.VMEM((_PER,), jnp.int32),
            pltpu.VMEM((16,), jnp.float32),
            pltpu.VMEM((16,), jnp.float32),
        ],
    )
    def _sc_loss(w_hbm, lx_hbm, ly_hbm, lz_hbm, idxn_hbm, idxs_hbm, out_hbm,
                 w_v, x_v, y_v, z_v, in_v, is_v, ow_v, ol_v):
        wid = lax.axis_index("s") * _NC + lax.axis_index("c")
        base = wid * _PER
        pltpu.sync_copy(w_hbm, w_v)
        pltpu.sync_copy(lx_hbm, x_v)
        pltpu.sync_copy(ly_hbm, y_v)
        pltpu.sync_copy(lz_hbm, z_v)
        pltpu.sync_copy(idxn_hbm.at[pl.ds(base, _PER)], in_v)
        pltpu.sync_copy(idxs_hbm.at[pl.ds(base, _PER)], is_v)

        def body(t, carry):
            accw, accl = carry
            off = t * 16
            jn = in_v[pl.ds(off, 16)]
            js = is_v[pl.ds(off, 16)]
            wj = plsc.load_gather(w_v, [jn])
            wi = plsc.load_gather(w_v, [js])
            dw = wi - wj
            accw = accw + dw * dw
            xj = plsc.load_gather(x_v, [jn])
            xi = plsc.load_gather(x_v, [js])
            dx = xi - xj
            accl = accl + dx * dx
            yj = plsc.load_gather(y_v, [jn])
            yi = plsc.load_gather(y_v, [js])
            dy = yi - yj
            accl = accl + dy * dy
            zj = plsc.load_gather(z_v, [jn])
            zi = plsc.load_gather(z_v, [js])
            dz = zi - zj
            accl = accl + dz * dz
            return accw, accl

        zero = jnp.zeros((16,), jnp.float32)
        accw, accl = lax.fori_loop(0, _CH, body, (zero, zero))
        ow_v[...] = accw
        ol_v[...] = accl
        pltpu.sync_copy(ow_v, out_hbm.at[wid, 0])
        pltpu.sync_copy(ol_v, out_hbm.at[wid, 1])

    return _sc_loss


def kernel(positions, weights, log_scales):
    idx = pl.pallas_call(
        _topk_body,
        grid=(N // ROWS,),
        in_specs=[
            pl.BlockSpec((ROWS, 3), lambda i: (i, 0)),
            pl.BlockSpec((3, N), lambda i: (0, 0)),
        ],
        out_specs=pl.BlockSpec((ROWS, K), lambda i: (i, 0)),
        out_shape=jax.ShapeDtypeStruct((N, K), jnp.int32),
        scratch_shapes=[pltpu.VMEM((ROWS, N), jnp.float32)],
    )(positions, positions.T)

    idxn = idx.reshape(-1)
    idxs = jnp.repeat(jnp.arange(N, dtype=jnp.int32), K)
    parts = _make_sc_loss()(
        weights,
        log_scales[:, 0],
        log_scales[:, 1],
        log_scales[:, 2],
        idxn,
        idxs,
    )
    sw = jnp.sum(parts[:, 0, :])
    sl = jnp.sum(parts[:, 1, :])
    loss = LAMBDA_S * (sw / (N * K) + sl / (N * K * 3))
    return loss.astype(jnp.float32)
